# baseline (speedup 1.0000x reference)
import functools

import jax
import jax.numpy as jnp
from jax import lax
from jax.experimental import pallas as pl
from jax.experimental.pallas import tpu as pltpu
from jax.experimental.pallas import tpu_sc as plsc

B = 16384
NPER = 16
N = B * NPER
NW = 32
COLS_W = N // NW
SEGS_W = B // NW
CCOLS = 2048
NCHUNK = COLS_W // CCOLS
CSEGS = CCOLS // NPER
NBUF = 2
TBL = 128


def _sc_body(e_hbm, x_hbm, t_hbm, tbl_hbm, o_hbm,
             e_v, x_v, o_v, t_v, tbl_v, s_v, in_sems, out_sems):
    wid = lax.axis_index("s") * 2 + lax.axis_index("c")
    tbase = wid * SEGS_W

    pltpu.sync_copy(tbl_hbm, tbl_v)
    pltpu.sync_copy(t_hbm.at[pl.ds(tbase, SEGS_W)], t_v)

    def in_slices(k):
        cb = wid * COLS_W + k * CCOLS
        b = k % NBUF
        return (
            (e_hbm.at[:, pl.ds(cb, CCOLS)], e_v.at[b]),
            (x_hbm.at[:, pl.ds(cb, CCOLS)], x_v.at[b]),
        )

    def out_slice(k):
        cb = wid * COLS_W + k * CCOLS
        return (o_v.at[k % NBUF], o_hbm.at[:, pl.ds(cb, CCOLS)])

    for k in range(NBUF):
        for src, dst in in_slices(k):
            pltpu.async_copy(src, dst, in_sems.at[k % NBUF])

    @plsc.parallel_loop(0, SEGS_W // 16, step=1, unroll=4)
    def scales(g):
        tv = t_v[pl.ds(g * 16, 16)]
        s_v[pl.ds(g * 16, 16)] = plsc.load_gather(tbl_v, [tv])

    iota = lax.iota(jnp.int32, 16)

    for k in range(NCHUNK):
        b = k % NBUF
        for src, dst in in_slices(k):
            pltpu.make_async_copy(src, dst, in_sems.at[b]).wait()
        if k >= NBUF:
            pltpu.make_async_copy(*out_slice(k - NBUF), out_sems.at[b]).wait()

        @plsc.parallel_loop(0, CSEGS, step=1, unroll=16)
        def seg(s):
            sv = plsc.load_gather(
                s_v, [jnp.full((16,), k * CSEGS, jnp.int32) + s])
            col = s * 16
            for c in range(3):
                tr = e_v[b, c, pl.ds(col, 16)] * sv
                m = jnp.sum(tr) * (1.0 / 16.0)
                r = x_v[b, c, pl.ds(col, 16)] + (tr - m)
                rr = r - r.astype(jnp.int32).astype(jnp.float32)
                o_v[b, c, pl.ds(col, 16)] = jnp.where(rr < 0, rr + 1.0, rr)

        pltpu.async_copy(*out_slice(k), out_sems.at[b])
        if k + NBUF < NCHUNK:
            for src, dst in in_slices(k + NBUF):
                pltpu.async_copy(src, dst, in_sems.at[b])

    for k in range(NCHUNK - NBUF, NCHUNK):
        pltpu.make_async_copy(*out_slice(k), out_sems.at[k % NBUF]).wait()


@jax.jit
def _run(e, x, t, tbl):
    mesh = plsc.VectorSubcoreMesh(core_axis_name="c", subcore_axis_name="s")
    f = pl.kernel(
        _sc_body,
        out_type=jax.ShapeDtypeStruct((3, N), jnp.float32),
        mesh=mesh,
        compiler_params=pltpu.CompilerParams(needs_layout_passes=False),
        scratch_types=[
            pltpu.VMEM((NBUF, 3, CCOLS), jnp.float32),
            pltpu.VMEM((NBUF, 3, CCOLS), jnp.float32),
            pltpu.VMEM((NBUF, 3, CCOLS), jnp.float32),
            pltpu.VMEM((SEGS_W,), jnp.int32),
            pltpu.VMEM((TBL,), jnp.float32),
            pltpu.VMEM((SEGS_W,), jnp.float32),
            pltpu.SemaphoreType.DMA((NBUF,)),
            pltpu.SemaphoreType.DMA((NBUF,)),
        ],
    )
    return f(e, x, t, tbl)


def kernel(x, t, num_atoms, epsilon, x_alphas_bar):
    del num_atoms
    tbl = jnp.sqrt(1.0 - x_alphas_bar)
    tbl = jnp.pad(tbl, (0, TBL - tbl.shape[0]))
    out = _run(epsilon.T, x.T, t, tbl)
    return out.T


# speedup vs baseline: 1.1655x; 1.1655x over previous
import functools

import jax
import jax.numpy as jnp
from jax import lax
from jax.experimental import pallas as pl
from jax.experimental.pallas import tpu as pltpu
from jax.experimental.pallas import tpu_sc as plsc

B = 16384
NPER = 16
N = B * NPER
NW = 32
COLS_W = N // NW
SEGS_W = B // NW
CCOLS = 2048
NCHUNK = COLS_W // CCOLS
CSEGS = CCOLS // NPER
NBUF = 2
TBL = 128


def _sc_body(e_hbm, x_hbm, t_hbm, tbl_hbm, o_hbm,
             e_v, x_v, o_v, t_v, tbl_v, s_v, in_sems, out_sems):
    wid = lax.axis_index("s") * 2 + lax.axis_index("c")
    tbase = wid * SEGS_W

    pltpu.sync_copy(tbl_hbm, tbl_v)
    pltpu.sync_copy(t_hbm.at[pl.ds(tbase, SEGS_W)], t_v)

    def in_slices(k):
        cb = wid * COLS_W + k * CCOLS
        b = k % NBUF
        return (
            (e_hbm.at[:, pl.ds(cb, CCOLS)], e_v.at[b]),
            (x_hbm.at[:, pl.ds(cb, CCOLS)], x_v.at[b]),
        )

    def out_slice(k):
        cb = wid * COLS_W + k * CCOLS
        return (o_v.at[k % NBUF], o_hbm.at[:, pl.ds(cb, CCOLS)])

    for k in range(NBUF):
        for src, dst in in_slices(k):
            pltpu.async_copy(src, dst, in_sems.at[k % NBUF])

    @plsc.parallel_loop(0, SEGS_W // 16, step=1, unroll=4)
    def scales(g):
        tv = t_v[pl.ds(g * 16, 16)]
        s_v[pl.ds(g * 16, 16)] = plsc.load_gather(tbl_v, [tv])

    iota = lax.iota(jnp.int32, 16)

    for k in range(NCHUNK):
        b = k % NBUF
        for src, dst in in_slices(k):
            pltpu.make_async_copy(src, dst, in_sems.at[b]).wait()
        if k >= NBUF:
            pltpu.make_async_copy(*out_slice(k - NBUF), out_sems.at[b]).wait()

        @plsc.parallel_loop(0, CSEGS, step=1, unroll=4)
        def seg(s):
            sv = plsc.load_gather(
                s_v, [jnp.full((16,), k * CSEGS, jnp.int32) + s])
            col = s * 16
            for c in range(3):
                tr = e_v[b, c, pl.ds(col, 16)] * sv
                m = jnp.sum(tr) * (1.0 / 16.0)
                r = x_v[b, c, pl.ds(col, 16)] + (tr - m)
                rr = r - r.astype(jnp.int32).astype(jnp.float32)
                o_v[b, c, pl.ds(col, 16)] = jnp.where(rr < 0, rr + 1.0, rr)

        pltpu.async_copy(*out_slice(k), out_sems.at[b])
        if k + NBUF < NCHUNK:
            for src, dst in in_slices(k + NBUF):
                pltpu.async_copy(src, dst, in_sems.at[b])

    for k in range(NCHUNK - NBUF, NCHUNK):
        pltpu.make_async_copy(*out_slice(k), out_sems.at[k % NBUF]).wait()


@jax.jit
def _run(e, x, t, tbl):
    mesh = plsc.VectorSubcoreMesh(core_axis_name="c", subcore_axis_name="s")
    f = pl.kernel(
        _sc_body,
        out_type=jax.ShapeDtypeStruct((3, N), jnp.float32),
        mesh=mesh,
        compiler_params=pltpu.CompilerParams(needs_layout_passes=False),
        scratch_types=[
            pltpu.VMEM((NBUF, 3, CCOLS), jnp.float32),
            pltpu.VMEM((NBUF, 3, CCOLS), jnp.float32),
            pltpu.VMEM((NBUF, 3, CCOLS), jnp.float32),
            pltpu.VMEM((SEGS_W,), jnp.int32),
            pltpu.VMEM((TBL,), jnp.float32),
            pltpu.VMEM((SEGS_W,), jnp.float32),
            pltpu.SemaphoreType.DMA((NBUF,)),
            pltpu.SemaphoreType.DMA((NBUF,)),
        ],
    )
    return f(e, x, t, tbl)


def kernel(x, t, num_atoms, epsilon, x_alphas_bar):
    del num_atoms
    tbl = jnp.sqrt(1.0 - x_alphas_bar)
    tbl = jnp.pad(tbl, (0, TBL - tbl.shape[0]))
    out = _run(epsilon.T, x.T, t, tbl)
    return out.T


# speedup vs baseline: 1.2405x; 1.0644x over previous
import functools

import jax
import jax.numpy as jnp
from jax import lax
from jax.experimental import pallas as pl
from jax.experimental.pallas import tpu as pltpu
from jax.experimental.pallas import tpu_sc as plsc

B = 16384
NPER = 16
N = B * NPER
NW = 32
COLS_W = N // NW
SEGS_W = B // NW
CCOLS = 2048
NCHUNK = COLS_W // CCOLS
CSEGS = CCOLS // NPER
NBUF = 2
TBL = 128


def _sc_body(e_hbm, x_hbm, t_hbm, tbl_hbm, o_hbm,
             e_v, x_v, o_v, t_v, tbl_v, s_v, in_sems, out_sems):
    wid = lax.axis_index("s") * 2 + lax.axis_index("c")
    tbase = wid * SEGS_W

    pltpu.sync_copy(tbl_hbm, tbl_v)
    pltpu.sync_copy(t_hbm.at[pl.ds(tbase, SEGS_W)], t_v)

    def in_slices(k):
        cb = wid * COLS_W + k * CCOLS
        b = k % NBUF
        return (
            (e_hbm.at[:, pl.ds(cb, CCOLS)], e_v.at[b]),
            (x_hbm.at[:, pl.ds(cb, CCOLS)], x_v.at[b]),
        )

    def out_slice(k):
        cb = wid * COLS_W + k * CCOLS
        return (o_v.at[k % NBUF], o_hbm.at[:, pl.ds(cb, CCOLS)])

    for k in range(NBUF):
        for src, dst in in_slices(k):
            pltpu.async_copy(src, dst, in_sems.at[k % NBUF])

    @plsc.parallel_loop(0, SEGS_W // 16, step=1, unroll=4)
    def scales(g):
        tv = t_v[pl.ds(g * 16, 16)]
        s_v[pl.ds(g * 16, 16)] = plsc.load_gather(tbl_v, [tv])

    iota = lax.iota(jnp.int32, 16)

    for k in range(NCHUNK):
        b = k % NBUF
        for src, dst in in_slices(k):
            pltpu.make_async_copy(src, dst, in_sems.at[b]).wait()
        if k >= NBUF:
            pltpu.make_async_copy(*out_slice(k - NBUF), out_sems.at[b]).wait()

        @plsc.parallel_loop(0, CSEGS, step=1, unroll=2)
        def seg(s):
            sv = plsc.load_gather(
                s_v, [jnp.full((16,), k * CSEGS, jnp.int32) + s])
            col = s * 16
            for c in range(3):
                tr = e_v[b, c, pl.ds(col, 16)] * sv
                m = jnp.sum(tr) * (1.0 / 16.0)
                r = x_v[b, c, pl.ds(col, 16)] + (tr - m)
                rr = r - r.astype(jnp.int32).astype(jnp.float32)
                o_v[b, c, pl.ds(col, 16)] = jnp.where(rr < 0, rr + 1.0, rr)

        pltpu.async_copy(*out_slice(k), out_sems.at[b])
        if k + NBUF < NCHUNK:
            for src, dst in in_slices(k + NBUF):
                pltpu.async_copy(src, dst, in_sems.at[b])

    for k in range(NCHUNK - NBUF, NCHUNK):
        pltpu.make_async_copy(*out_slice(k), out_sems.at[k % NBUF]).wait()


@jax.jit
def _run(e, x, t, tbl):
    mesh = plsc.VectorSubcoreMesh(core_axis_name="c", subcore_axis_name="s")
    f = pl.kernel(
        _sc_body,
        out_type=jax.ShapeDtypeStruct((3, N), jnp.float32),
        mesh=mesh,
        compiler_params=pltpu.CompilerParams(needs_layout_passes=False),
        scratch_types=[
            pltpu.VMEM((NBUF, 3, CCOLS), jnp.float32),
            pltpu.VMEM((NBUF, 3, CCOLS), jnp.float32),
            pltpu.VMEM((NBUF, 3, CCOLS), jnp.float32),
            pltpu.VMEM((SEGS_W,), jnp.int32),
            pltpu.VMEM((TBL,), jnp.float32),
            pltpu.VMEM((SEGS_W,), jnp.float32),
            pltpu.SemaphoreType.DMA((NBUF,)),
            pltpu.SemaphoreType.DMA((NBUF,)),
        ],
    )
    return f(e, x, t, tbl)


def kernel(x, t, num_atoms, epsilon, x_alphas_bar):
    del num_atoms
    tbl = jnp.sqrt(1.0 - x_alphas_bar)
    tbl = jnp.pad(tbl, (0, TBL - tbl.shape[0]))
    out = _run(epsilon.T, x.T, t, tbl)
    return out.T


# speedup vs baseline: 1.2500x; 1.0077x over previous
import functools

import jax
import jax.numpy as jnp
from jax import lax
from jax.experimental import pallas as pl
from jax.experimental.pallas import tpu as pltpu
from jax.experimental.pallas import tpu_sc as plsc

B = 16384
NPER = 16
N = B * NPER
NW = 32
COLS_W = N // NW
SEGS_W = B // NW
CCOLS = 2048
NCHUNK = COLS_W // CCOLS
CSEGS = CCOLS // NPER
NBUF = 2
TBL = 128


def _sc_body(e_hbm, x_hbm, t_hbm, tbl_hbm, o_hbm,
             e_v, x_v, o_v, t_v, tbl_v, s_v, in_sems, out_sems):
    wid = lax.axis_index("s") * 2 + lax.axis_index("c")
    tbase = wid * SEGS_W

    pltpu.sync_copy(tbl_hbm, tbl_v)
    pltpu.sync_copy(t_hbm.at[pl.ds(tbase, SEGS_W)], t_v)

    def in_slices(k):
        cb = wid * COLS_W + k * CCOLS
        b = k % NBUF
        return (
            (e_hbm.at[:, pl.ds(cb, CCOLS)], e_v.at[b]),
            (x_hbm.at[:, pl.ds(cb, CCOLS)], x_v.at[b]),
        )

    def out_slice(k):
        cb = wid * COLS_W + k * CCOLS
        return (o_v.at[k % NBUF], o_hbm.at[:, pl.ds(cb, CCOLS)])

    for k in range(NBUF):
        for src, dst in in_slices(k):
            pltpu.async_copy(src, dst, in_sems.at[k % NBUF])

    @plsc.parallel_loop(0, SEGS_W // 16, step=1, unroll=4)
    def scales(g):
        tv = t_v[pl.ds(g * 16, 16)]
        s_v[pl.ds(g * 16, 16)] = plsc.load_gather(tbl_v, [tv])

    iota = lax.iota(jnp.int32, 16)

    for k in range(NCHUNK):
        b = k % NBUF
        for src, dst in in_slices(k):
            pltpu.make_async_copy(src, dst, in_sems.at[b]).wait()
        if k >= NBUF:
            pltpu.make_async_copy(*out_slice(k - NBUF), out_sems.at[b]).wait()

        @plsc.parallel_loop(0, CSEGS, step=1, unroll=1)
        def seg(s):
            sv = plsc.load_gather(
                s_v, [jnp.full((16,), k * CSEGS, jnp.int32) + s])
            col = s * 16
            for c in range(3):
                tr = e_v[b, c, pl.ds(col, 16)] * sv
                m = jnp.sum(tr) * (1.0 / 16.0)
                r = x_v[b, c, pl.ds(col, 16)] + (tr - m)
                rr = r - r.astype(jnp.int32).astype(jnp.float32)
                o_v[b, c, pl.ds(col, 16)] = jnp.where(rr < 0, rr + 1.0, rr)

        pltpu.async_copy(*out_slice(k), out_sems.at[b])
        if k + NBUF < NCHUNK:
            for src, dst in in_slices(k + NBUF):
                pltpu.async_copy(src, dst, in_sems.at[b])

    for k in range(NCHUNK - NBUF, NCHUNK):
        pltpu.make_async_copy(*out_slice(k), out_sems.at[k % NBUF]).wait()


@jax.jit
def _run(e, x, t, tbl):
    mesh = plsc.VectorSubcoreMesh(core_axis_name="c", subcore_axis_name="s")
    f = pl.kernel(
        _sc_body,
        out_type=jax.ShapeDtypeStruct((3, N), jnp.float32),
        mesh=mesh,
        compiler_params=pltpu.CompilerParams(needs_layout_passes=False),
        scratch_types=[
            pltpu.VMEM((NBUF, 3, CCOLS), jnp.float32),
            pltpu.VMEM((NBUF, 3, CCOLS), jnp.float32),
            pltpu.VMEM((NBUF, 3, CCOLS), jnp.float32),
            pltpu.VMEM((SEGS_W,), jnp.int32),
            pltpu.VMEM((TBL,), jnp.float32),
            pltpu.VMEM((SEGS_W,), jnp.float32),
            pltpu.SemaphoreType.DMA((NBUF,)),
            pltpu.SemaphoreType.DMA((NBUF,)),
        ],
    )
    return f(e, x, t, tbl)


def kernel(x, t, num_atoms, epsilon, x_alphas_bar):
    del num_atoms
    tbl = jnp.sqrt(1.0 - x_alphas_bar)
    tbl = jnp.pad(tbl, (0, TBL - tbl.shape[0]))
    out = _run(epsilon.T, x.T, t, tbl)
    return out.T
